# baseline (speedup 1.0000x reference)
import functools

import jax
import jax.numpy as jnp
from jax import lax
from jax.experimental import pallas as pl
from jax.experimental.pallas import tpu as pltpu
from jax.experimental.pallas import tpu_sc as plsc

F32 = jnp.float32

_N = 10000
_E = 320000
_D = 128
_G = 16

_NC = 2
_NS = 16
_NW = _NC * _NS
_EW = _E // _NW
_CH = 125
_NCHROWS = _E // _CH
_CPW = _NCHROWS // _NW
_NBUF = 4
_NP = 10240
_RW = _NP // _NS
_ZR = 32

_BLK = 1000
_NBLK = _N // _BLK


def _deg_body(dst_hbm, out_hbm, dstv, ones_b, zb, acc, gsem):
    c = lax.axis_index("c")
    s = lax.axis_index("s")
    wid = s * _NC + c
    one = jnp.ones((16,), F32)
    zero = jnp.zeros((16,), F32)
    for i in range(_CH):
        ones_b[i] = one
    for i in range(_ZR):
        zb[i] = zero
    def zloop(t, carry):
        pltpu.sync_copy(zb, acc.at[pl.ds(s * _RW + t * _ZR, _ZR)])
        return carry
    lax.fori_loop(0, _RW // _ZR, zloop, 0)
    pltpu.sync_copy(dst_hbm.at[pl.ds(wid * _CPW, _CPW)], dstv)
    plsc.subcore_barrier()
    def eloop(t, carry):
        pltpu.sync_copy(ones_b, acc.at[dstv.at[t]], add=True)
        return carry
    lax.fori_loop(0, _CPW, eloop, 0)
    plsc.subcore_barrier()
    pltpu.sync_copy(acc.at[pl.ds(s * _RW, _RW)],
                    out_hbm.at[c, pl.ds(s * _RW, _RW)])


_deg_call = pl.kernel(
    _deg_body,
    out_type=jax.ShapeDtypeStruct((_NC, _NP, 16), F32),
    mesh=plsc.VectorSubcoreMesh(core_axis_name="c", subcore_axis_name="s"),
    scratch_types=[
        pltpu.VMEM((_CPW, _CH), jnp.int32),
        pltpu.VMEM((_CH, 16), F32),
        pltpu.VMEM((_ZR, 16), F32),
        pltpu.VMEM_SHARED((_NP, 16), F32),
        pltpu.SemaphoreType.DMA,
    ],
)


def _agg_body(hs_hbm, e3_hbm, out_hbm, ib0, ib1, r0, r1, zb, acc,
              is0, is1, gs0, gs1, ss0, ss1):
    ibs = (ib0, ib1)
    isems = (is0, is1)
    rows = (r0, r1)
    gsems = (gs0, gs1)
    ssems = (ss0, ss1)
    c = lax.axis_index("c")
    s = lax.axis_index("s")
    wid = s * _NC + c
    zero = jnp.zeros((16,), F32)
    for i in range(_ZR):
        for j in range(_D // 16):
            zb[i, pl.ds(16 * j, 16)] = zero
    def zloop(t, carry):
        pltpu.sync_copy(zb, acc.at[pl.ds(s * _RW + t * _ZR, _ZR)])
        return carry
    lax.fori_loop(0, _RW // _ZR, zloop, 0)
    row0 = wid * _CPW
    pltpu.async_copy(e3_hbm.at[:, pl.ds(row0, 8)], ib0, is0)
    plsc.subcore_barrier()
    pltpu.make_async_copy(e3_hbm.at[:, pl.ds(row0, 8)], ib0, is0).wait()
    pltpu.async_copy(hs_hbm.at[ib0.at[0, 0]], r0, gs0)

    def group(g, cur, nxt, ci, ni, first):
        for d in range(8):
            b = d % 2
            nb = 1 - b
            if first and d == 0:
                pass
            else:
                pltpu.make_async_copy(
                    rows[nb],
                    acc.at[(cur if d != 0 else nxt).at[1, (d + 7) % 8]],
                    ssems[nb]).wait()
            if d == 0:
                gnxt = row0 + jnp.minimum((g + 1) * 8, (_CPW // 8 - 1) * 8)
                pltpu.async_copy(e3_hbm.at[:, pl.ds(gnxt, 8)], ibs[ni],
                                 isems[ni])
            if d == 7:
                pltpu.make_async_copy(e3_hbm.at[:, pl.ds(row0, 8)], ibs[ni],
                                      isems[ni]).wait()
                pltpu.async_copy(hs_hbm.at[ibs[ni].at[0, 0]], rows[nb],
                                 gsems[nb])
            else:
                pltpu.async_copy(hs_hbm.at[cur.at[0, d + 1]], rows[nb],
                                 gsems[nb])
            pltpu.make_async_copy(hs_hbm.at[cur.at[0, d]], rows[b],
                                  gsems[b]).wait()
            pltpu.async_copy(rows[b], acc.at[cur.at[1, d]], ssems[b],
                             add=True)

    group(0, ib0, ib1, 0, 1, True)
    def gloop(j, carry):
        g = 1 + j * 2
        group(g, ib1, ib0, 1, 0, False)
        group(g + 1, ib0, ib1, 0, 1, False)
        return carry
    lax.fori_loop(0, (_CPW // 8 - 1) // 2, gloop, 0, unroll=False)
    group(_CPW // 8 - 1, ib1, ib0, 1, 0, False)
    pltpu.make_async_copy(hs_hbm.at[ib0.at[0, 0]], r0, gs0).wait()
    pltpu.make_async_copy(rows[1], acc.at[ib1.at[1, 7]], ss1).wait()
    plsc.subcore_barrier()
    pltpu.sync_copy(acc.at[pl.ds(s * _RW, _RW)],
                    out_hbm.at[c, pl.ds(s * _RW, _RW)])


_agg_call = pl.kernel(
    _agg_body,
    out_type=jax.ShapeDtypeStruct((_NC, _NP, _D), F32),
    mesh=plsc.VectorSubcoreMesh(core_axis_name="c", subcore_axis_name="s"),
    scratch_types=(
        [pltpu.VMEM((2, 8, _CH), jnp.int32) for _ in range(2)]
        + [pltpu.VMEM((_CH, _D), F32) for _ in range(2)]
        + [pltpu.VMEM((_ZR, _D), F32),
           pltpu.VMEM_SHARED((_NP, _D), F32)]
        + [pltpu.SemaphoreType.DMA for _ in range(6)]
    ),
)


def _pre_body(degp_ref, x_ref, w_ref, hs_ref, dinv_ref):
    dp = degp_ref[0] + degp_ref[1]
    dinv = lax.rsqrt(1.0 + dp[:, 0:1])
    h = jnp.dot(x_ref[:], w_ref[:], preferred_element_type=F32)
    hs_ref[:] = h * dinv
    dinv_ref[:] = jnp.broadcast_to(dinv, (_BLK, _D))


_pre_call = pl.pallas_call(
    _pre_body,
    grid=(_NBLK,),
    in_specs=[
        pl.BlockSpec((_NC, _BLK, 16), lambda i: (0, i, 0)),
        pl.BlockSpec((_BLK, _D), lambda i: (i, 0)),
        pl.BlockSpec((_D, _D), lambda i: (0, 0)),
    ],
    out_specs=[
        pl.BlockSpec((_BLK, _D), lambda i: (i, 0)),
        pl.BlockSpec((_BLK, _D), lambda i: (i, 0)),
    ],
    out_shape=[
        jax.ShapeDtypeStruct((_N, _D), F32),
        jax.ShapeDtypeStruct((_N, _D), F32),
    ],
)


def _post_body(s_ref, hs_ref, dinv_ref, b_ref, z_ref, sum_ref, sq_ref):
    z = dinv_ref[:] * (s_ref[0] + s_ref[1] + hs_ref[:]) + b_ref[:]
    z_ref[:] = z

    @pl.when(pl.program_id(0) == 0)
    def _():
        sum_ref[:] = jnp.zeros_like(sum_ref)
        sq_ref[:] = jnp.zeros_like(sq_ref)

    sum_ref[:] = sum_ref[:] + jnp.sum(z, axis=0, keepdims=True)
    sq_ref[:] = sq_ref[:] + jnp.sum(z * z, axis=0, keepdims=True)


_post_call = pl.pallas_call(
    _post_body,
    grid=(_NBLK,),
    in_specs=[
        pl.BlockSpec((_NC, _BLK, _D), lambda i: (0, i, 0)),
        pl.BlockSpec((_BLK, _D), lambda i: (i, 0)),
        pl.BlockSpec((_BLK, _D), lambda i: (i, 0)),
        pl.BlockSpec((1, _D), lambda i: (0, 0)),
    ],
    out_specs=[
        pl.BlockSpec((_BLK, _D), lambda i: (i, 0)),
        pl.BlockSpec((1, _D), lambda i: (0, 0)),
        pl.BlockSpec((1, _D), lambda i: (0, 0)),
    ],
    out_shape=[
        jax.ShapeDtypeStruct((_N, _D), F32),
        jax.ShapeDtypeStruct((1, _D), F32),
        jax.ShapeDtypeStruct((1, _D), F32),
    ],
)


def _next_body(z_ref, sum_ref, sq_ref, g_ref, be_ref, w_ref, dinv_ref, o_ref):
    m = sum_ref[:] * (1.0 / _N)
    v = sq_ref[:] * (1.0 / _N) - m * m
    zn = (z_ref[:] - m) * lax.rsqrt(v + 1e-5) * g_ref[:] + be_ref[:]
    h = jnp.dot(jnp.maximum(zn, 0.0), w_ref[:], preferred_element_type=F32)
    o_ref[:] = h * dinv_ref[:]


_next_call = pl.pallas_call(
    _next_body,
    grid=(_NBLK,),
    in_specs=[
        pl.BlockSpec((_BLK, _D), lambda i: (i, 0)),
        pl.BlockSpec((1, _D), lambda i: (0, 0)),
        pl.BlockSpec((1, _D), lambda i: (0, 0)),
        pl.BlockSpec((1, _D), lambda i: (0, 0)),
        pl.BlockSpec((1, _D), lambda i: (0, 0)),
        pl.BlockSpec((_D, _D), lambda i: (0, 0)),
        pl.BlockSpec((_BLK, _D), lambda i: (i, 0)),
    ],
    out_specs=pl.BlockSpec((_BLK, _D), lambda i: (i, 0)),
    out_shape=jax.ShapeDtypeStruct((_N, _D), F32),
)


def _gate_of(z, gw1_ref, gb1_ref, gw2_ref, gb2_ref):
    r = jnp.maximum(
        jnp.dot(z, gw1_ref[:], preferred_element_type=F32) + gb1_ref[:], 0.0)
    return jnp.sum(r * gw2_ref[:], axis=1, keepdims=True) + gb2_ref[:]


def _post3_body(s_ref, hs_ref, dinv_ref, b_ref, gw1_ref, gb1_ref, gw2_ref,
                gb2_ref, batch_ref, z_ref, gmax_ref):
    z = dinv_ref[:] * (s_ref[0] + s_ref[1] + hs_ref[:]) + b_ref[:]
    z_ref[:] = z
    gate = _gate_of(z, gw1_ref, gb1_ref, gw2_ref, gb2_ref)
    lane = lax.broadcasted_iota(jnp.int32, (_BLK, _D), 1)
    mask = batch_ref[:] == lane
    neg = jnp.full((_BLK, _D), -jnp.inf, F32)
    masked = jnp.where(mask, jnp.broadcast_to(gate, (_BLK, _D)), neg)
    bmax = jnp.max(masked, axis=0, keepdims=True)

    @pl.when(pl.program_id(0) == 0)
    def _():
        gmax_ref[:] = jnp.full((1, _D), -jnp.inf, F32)

    gmax_ref[:] = jnp.maximum(gmax_ref[:], bmax)


_post3_call = pl.pallas_call(
    _post3_body,
    grid=(_NBLK,),
    in_specs=[
        pl.BlockSpec((_NC, _BLK, _D), lambda i: (0, i, 0)),
        pl.BlockSpec((_BLK, _D), lambda i: (i, 0)),
        pl.BlockSpec((_BLK, _D), lambda i: (i, 0)),
        pl.BlockSpec((1, _D), lambda i: (0, 0)),
        pl.BlockSpec((_D, _D), lambda i: (0, 0)),
        pl.BlockSpec((1, _D), lambda i: (0, 0)),
        pl.BlockSpec((1, _D), lambda i: (0, 0)),
        pl.BlockSpec((1, 1), lambda i: (0, 0)),
        pl.BlockSpec((_BLK, _D), lambda i: (i, 0)),
    ],
    out_specs=[
        pl.BlockSpec((_BLK, _D), lambda i: (i, 0)),
        pl.BlockSpec((1, _D), lambda i: (0, 0)),
    ],
    out_shape=[
        jax.ShapeDtypeStruct((_N, _D), F32),
        jax.ShapeDtypeStruct((1, _D), F32),
    ],
)


def _fin_body(z_ref, gmax_ref, batch_ref, gw1_ref, gb1_ref, gw2_ref, gb2_ref,
              mw1_ref, mb1_ref, mw2_ref, mb2_ref, o_ref, num_scr, den_scr):
    i = pl.program_id(0)

    @pl.when(i == 0)
    def _():
        num_scr[:] = jnp.zeros_like(num_scr)
        den_scr[:] = jnp.zeros_like(den_scr)

    z = z_ref[:]
    gate = _gate_of(z, gw1_ref, gb1_ref, gw2_ref, gb2_ref)
    g = gmax_ref[:]
    mxf = jnp.where(g > -jnp.inf, g, 0.0)
    lane = lax.broadcasted_iota(jnp.int32, (_BLK, _D), 1)
    maskf = (batch_ref[:] == lane).astype(F32)
    mxb = jnp.sum(maskf * mxf, axis=1, keepdims=True)
    e = jnp.exp(gate - mxb)
    contrib = maskf * e
    dn = (((0,), (0,)), ((), ()))
    num_scr[:] = num_scr[:] + lax.dot_general(
        contrib, z, dn, preferred_element_type=F32)
    den_scr[:] = den_scr[:] + lax.dot_general(
        contrib, jnp.ones((_BLK, _D), F32), dn, preferred_element_type=F32)

    @pl.when(i == pl.num_programs(0) - 1)
    def _():
        pooled = num_scr[:] / (den_scr[:] + 1e-16)
        t = jnp.maximum(
            jnp.dot(pooled, mw1_ref[:], preferred_element_type=F32)
            + mb1_ref[:], 0.0)
        res = jnp.dot(t, mw2_ref[:], preferred_element_type=F32) + mb2_ref[:]
        o_ref[:] = res[0:_G, :]


_fin_call = pl.pallas_call(
    _fin_body,
    grid=(_NBLK,),
    in_specs=[
        pl.BlockSpec((_BLK, _D), lambda i: (i, 0)),
        pl.BlockSpec((1, _D), lambda i: (0, 0)),
        pl.BlockSpec((_BLK, _D), lambda i: (i, 0)),
        pl.BlockSpec((_D, _D), lambda i: (0, 0)),
        pl.BlockSpec((1, _D), lambda i: (0, 0)),
        pl.BlockSpec((1, _D), lambda i: (0, 0)),
        pl.BlockSpec((1, 1), lambda i: (0, 0)),
        pl.BlockSpec((_D, _D), lambda i: (0, 0)),
        pl.BlockSpec((1, _D), lambda i: (0, 0)),
        pl.BlockSpec((_D, _D), lambda i: (0, 0)),
        pl.BlockSpec((1, _D), lambda i: (0, 0)),
    ],
    out_specs=pl.BlockSpec((_G, _D), lambda i: (0, 0)),
    out_shape=jax.ShapeDtypeStruct((_G, _D), F32),
    scratch_shapes=[
        pltpu.VMEM((_D, _D), F32),
        pltpu.VMEM((_D, _D), F32),
    ],
)


def kernel(x, edge_index, batch, W1, b1, W2, b2, W3, b3, g1, be1, g2, be2,
           gW1, gb1, gW2, gb2, mW1, mb1, mW2, mb2):
    e3 = edge_index.reshape(_NC, _NCHROWS, _CH)
    dst = e3[1]
    batchb = jnp.broadcast_to(batch.astype(jnp.int32)[:, None], (_N, _D))
    b1r = b1.reshape(1, _D)
    b2r = b2.reshape(1, _D)
    b3r = b3.reshape(1, _D)
    g1r = g1.reshape(1, _D)
    be1r = be1.reshape(1, _D)
    g2r = g2.reshape(1, _D)
    be2r = be2.reshape(1, _D)
    gb1r = gb1.reshape(1, _D)
    gw2r = gW2.reshape(1, _D)
    gb2r = gb2.reshape(1, 1)
    mb1r = mb1.reshape(1, _D)
    mb2r = mb2.reshape(1, _D)

    degp = _deg_call(dst)
    hs1, dinvb = _pre_call(degp, x, W1)
    s1 = _agg_call(hs1, e3)
    z1, sm1, sq1 = _post_call(s1, hs1, dinvb, b1r)
    hs2 = _next_call(z1, sm1, sq1, g1r, be1r, W2, dinvb)
    s2 = _agg_call(hs2, e3)
    z2, sm2, sq2 = _post_call(s2, hs2, dinvb, b2r)
    hs3 = _next_call(z2, sm2, sq2, g2r, be2r, W3, dinvb)
    s3 = _agg_call(hs3, e3)
    z3, gmax = _post3_call(s3, hs3, dinvb, b3r, gW1, gb1r, gw2r, gb2r, batchb)
    out = _fin_call(z3, gmax, batchb, gW1, gb1r, gw2r, gb2r, mW1, mb1r, mW2,
                    mb2r)
    return out


# speedup vs baseline: 1.0105x; 1.0105x over previous
import functools

import jax
import jax.numpy as jnp
from jax import lax
from jax.experimental import pallas as pl
from jax.experimental.pallas import tpu as pltpu
from jax.experimental.pallas import tpu_sc as plsc

F32 = jnp.float32

_N = 10000
_E = 320000
_D = 128
_G = 16

_NC = 2
_NS = 16
_NW = _NC * _NS
_EW = _E // _NW
_CH = 125
_NCHROWS = _E // _CH
_CPW = _NCHROWS // _NW
_NBUF = 4
_NP = 10240
_RW = _NP // _NS
_ZR = 32

_BLK = 1000
_NBLK = _N // _BLK


def _deg_body(dst_hbm, out_hbm, dstv, ones_b, zb, acc, gsem):
    c = lax.axis_index("c")
    s = lax.axis_index("s")
    wid = s * _NC + c
    one = jnp.ones((16,), F32)
    zero = jnp.zeros((16,), F32)
    for i in range(_CH):
        ones_b[i] = one
    for i in range(_ZR):
        zb[i] = zero
    def zloop(t, carry):
        pltpu.sync_copy(zb, acc.at[pl.ds(s * _RW + t * _ZR, _ZR)])
        return carry
    lax.fori_loop(0, _RW // _ZR, zloop, 0)
    pltpu.sync_copy(dst_hbm.at[pl.ds(wid * _CPW, _CPW)], dstv)
    plsc.subcore_barrier()
    def eloop(t, carry):
        pltpu.sync_copy(ones_b, acc.at[dstv.at[t]], add=True)
        return carry
    lax.fori_loop(0, _CPW, eloop, 0)
    plsc.subcore_barrier()
    pltpu.sync_copy(acc.at[pl.ds(s * _RW, _RW)],
                    out_hbm.at[c, pl.ds(s * _RW, _RW)])


_deg_call = pl.kernel(
    _deg_body,
    out_type=jax.ShapeDtypeStruct((_NC, _NP, 16), F32),
    mesh=plsc.VectorSubcoreMesh(core_axis_name="c", subcore_axis_name="s"),
    scratch_types=[
        pltpu.VMEM((_CPW, _CH), jnp.int32),
        pltpu.VMEM((_CH, 16), F32),
        pltpu.VMEM((_ZR, 16), F32),
        pltpu.VMEM_SHARED((_NP, 16), F32),
        pltpu.SemaphoreType.DMA,
    ],
)


def _agg_body(hs_hbm, e3_hbm, out_hbm, eb0, eb1, eb2, eb3, eb4, eb5, eb6,
              eb7, r0, r1, zb, acc, es0, es1, es2, es3, es4, es5, es6, es7,
              gs0, gs1, ss0, ss1):
    ebs = (eb0, eb1, eb2, eb3, eb4, eb5, eb6, eb7)
    esems = (es0, es1, es2, es3, es4, es5, es6, es7)
    rows = (r0, r1)
    gsems = (gs0, gs1)
    ssems = (ss0, ss1)
    c = lax.axis_index("c")
    s = lax.axis_index("s")
    wid = s * _NC + c
    zero = jnp.zeros((16,), F32)
    for i in range(_ZR):
        for j in range(_D // 16):
            zb[i, pl.ds(16 * j, 16)] = zero
    def zloop(t, carry):
        pltpu.sync_copy(zb, acc.at[pl.ds(s * _RW + t * _ZR, _ZR)])
        return carry
    lax.fori_loop(0, _RW // _ZR, zloop, 0)
    row0 = wid * _CPW
    for k in range(7):
        pltpu.async_copy(e3_hbm.at[:, row0 + k], ebs[k], esems[k])
    plsc.subcore_barrier()
    pltpu.make_async_copy(e3_hbm.at[:, row0], eb0, es0).wait()
    pltpu.async_copy(hs_hbm.at[eb0.at[0]], r0, gs0)

    def step(t, d, skip_swait):
        b = d % 2
        nb = 1 - b
        e1 = (d + 1) % 8
        e7 = (d + 7) % 8
        pltpu.make_async_copy(e3_hbm.at[:, row0], ebs[e1], esems[e1]).wait()
        if not skip_swait:
            pltpu.make_async_copy(rows[nb], acc.at[ebs[e7].at[1]],
                                  ssems[nb]).wait()
        pltpu.async_copy(hs_hbm.at[ebs[e1].at[0]], rows[nb], gsems[nb])
        pltpu.make_async_copy(hs_hbm.at[ebs[d].at[0]], rows[b],
                              gsems[b]).wait()
        pltpu.async_copy(rows[b], acc.at[ebs[d].at[1]], ssems[b], add=True)
        tn = row0 + jnp.minimum(t + 7, _CPW - 1)
        pltpu.async_copy(e3_hbm.at[:, tn], ebs[e7], esems[e7])

    for d in range(8):
        step(d, d, skip_swait=(d == 0))

    def eloop(g, carry):
        for d in range(8):
            step(g * 8 + d, d, skip_swait=False)
        return carry
    lax.fori_loop(1, _CPW // 8, eloop, 0)
    pltpu.make_async_copy(hs_hbm.at[eb0.at[0]], r0, gs0).wait()
    pltpu.make_async_copy(rows[1], acc.at[eb7.at[1]], ss1).wait()
    for k in range(1, 7):
        pltpu.make_async_copy(e3_hbm.at[:, row0], ebs[k], esems[k]).wait()
    plsc.subcore_barrier()
    pltpu.sync_copy(acc.at[pl.ds(s * _RW, _RW)],
                    out_hbm.at[c, pl.ds(s * _RW, _RW)])


_agg_call = pl.kernel(
    _agg_body,
    out_type=jax.ShapeDtypeStruct((_NC, _NP, _D), F32),
    mesh=plsc.VectorSubcoreMesh(core_axis_name="c", subcore_axis_name="s"),
    scratch_types=(
        [pltpu.VMEM((2, _CH), jnp.int32) for _ in range(8)]
        + [pltpu.VMEM((_CH, _D), F32) for _ in range(2)]
        + [pltpu.VMEM((_ZR, _D), F32),
           pltpu.VMEM_SHARED((_NP, _D), F32)]
        + [pltpu.SemaphoreType.DMA for _ in range(12)]
    ),
)


def _pre_body(degp_ref, x_ref, w_ref, hs_ref, dinv_ref):
    dp = degp_ref[0] + degp_ref[1]
    dinv = lax.rsqrt(1.0 + dp[:, 0:1])
    h = jnp.dot(x_ref[:], w_ref[:], preferred_element_type=F32)
    hs_ref[:] = h * dinv
    dinv_ref[:] = jnp.broadcast_to(dinv, (_BLK, _D))


_pre_call = pl.pallas_call(
    _pre_body,
    grid=(_NBLK,),
    in_specs=[
        pl.BlockSpec((_NC, _BLK, 16), lambda i: (0, i, 0)),
        pl.BlockSpec((_BLK, _D), lambda i: (i, 0)),
        pl.BlockSpec((_D, _D), lambda i: (0, 0)),
    ],
    out_specs=[
        pl.BlockSpec((_BLK, _D), lambda i: (i, 0)),
        pl.BlockSpec((_BLK, _D), lambda i: (i, 0)),
    ],
    out_shape=[
        jax.ShapeDtypeStruct((_N, _D), F32),
        jax.ShapeDtypeStruct((_N, _D), F32),
    ],
)


def _postnext_body(s_ref, hs_ref, dinv_ref, b_ref, g_ref, be_ref, w_ref,
                   o_ref, sum_scr, sq_scr):
    p = pl.program_id(0)
    i = pl.program_id(1)
    z = dinv_ref[:] * (s_ref[0] + s_ref[1] + hs_ref[:]) + b_ref[:]

    @pl.when((p == 0) & (i == 0))
    def _():
        sum_scr[:] = jnp.zeros_like(sum_scr)
        sq_scr[:] = jnp.zeros_like(sq_scr)

    @pl.when(p == 0)
    def _():
        sum_scr[:] = sum_scr[:] + jnp.sum(z, axis=0, keepdims=True)
        sq_scr[:] = sq_scr[:] + jnp.sum(z * z, axis=0, keepdims=True)
        o_ref[:] = z

    @pl.when(p == 1)
    def _():
        m = sum_scr[:] * (1.0 / _N)
        v = sq_scr[:] * (1.0 / _N) - m * m
        zn = (z - m) * lax.rsqrt(v + 1e-5) * g_ref[:] + be_ref[:]
        h = jnp.dot(jnp.maximum(zn, 0.0), w_ref[:],
                    preferred_element_type=F32)
        o_ref[:] = h * dinv_ref[:]


_postnext_call = pl.pallas_call(
    _postnext_body,
    grid=(2, _NBLK),
    in_specs=[
        pl.BlockSpec((_NC, _BLK, _D), lambda p, i: (0, i, 0)),
        pl.BlockSpec((_BLK, _D), lambda p, i: (i, 0)),
        pl.BlockSpec((_BLK, _D), lambda p, i: (i, 0)),
        pl.BlockSpec((1, _D), lambda p, i: (0, 0)),
        pl.BlockSpec((1, _D), lambda p, i: (0, 0)),
        pl.BlockSpec((1, _D), lambda p, i: (0, 0)),
        pl.BlockSpec((_D, _D), lambda p, i: (0, 0)),
    ],
    out_specs=pl.BlockSpec((_BLK, _D), lambda p, i: (i, 0)),
    out_shape=jax.ShapeDtypeStruct((_N, _D), F32),
    scratch_shapes=[
        pltpu.VMEM((1, _D), F32),
        pltpu.VMEM((1, _D), F32),
    ],
)


def _gate_of(z, gw1_ref, gb1_ref, gw2_ref, gb2_ref):
    r = jnp.maximum(
        jnp.dot(z, gw1_ref[:], preferred_element_type=F32) + gb1_ref[:], 0.0)
    return jnp.sum(r * gw2_ref[:], axis=1, keepdims=True) + gb2_ref[:]


def _fin3_body(s_ref, hs_ref, dinv_ref, b_ref, gw1_ref, gb1_ref, gw2_ref,
               gb2_ref, batch_ref, mw1_ref, mb1_ref, mw2_ref, mb2_ref,
               o_ref, gmax_scr, num_scr, den_scr):
    p = pl.program_id(0)
    i = pl.program_id(1)
    z = dinv_ref[:] * (s_ref[0] + s_ref[1] + hs_ref[:]) + b_ref[:]
    gate = _gate_of(z, gw1_ref, gb1_ref, gw2_ref, gb2_ref)
    lane = lax.broadcasted_iota(jnp.int32, (_BLK, _D), 1)
    mask = batch_ref[:] == lane

    @pl.when((p == 0) & (i == 0))
    def _():
        gmax_scr[:] = jnp.full((1, _D), -jnp.inf, F32)

    @pl.when(p == 0)
    def _():
        neg = jnp.full((_BLK, _D), -jnp.inf, F32)
        masked = jnp.where(mask, jnp.broadcast_to(gate, (_BLK, _D)), neg)
        gmax_scr[:] = jnp.maximum(gmax_scr[:],
                                  jnp.max(masked, axis=0, keepdims=True))

    @pl.when(p == 1)
    def _():
        @pl.when(i == 0)
        def _():
            num_scr[:] = jnp.zeros_like(num_scr)
            den_scr[:] = jnp.zeros_like(den_scr)

        g = gmax_scr[:]
        mxf = jnp.where(g > -jnp.inf, g, 0.0)
        maskf = mask.astype(F32)
        mxb = jnp.sum(maskf * mxf, axis=1, keepdims=True)
        e = jnp.exp(gate - mxb)
        contrib = maskf * e
        dn = (((0,), (0,)), ((), ()))
        num_scr[:] = num_scr[:] + lax.dot_general(
            contrib, z, dn, preferred_element_type=F32)
        den_scr[:] = den_scr[:] + lax.dot_general(
            contrib, jnp.ones((_BLK, _D), F32), dn,
            preferred_element_type=F32)

        @pl.when(i == _NBLK - 1)
        def _():
            pooled = num_scr[:] / (den_scr[:] + 1e-16)
            t = jnp.maximum(
                jnp.dot(pooled, mw1_ref[:], preferred_element_type=F32)
                + mb1_ref[:], 0.0)
            res = jnp.dot(t, mw2_ref[:],
                          preferred_element_type=F32) + mb2_ref[:]
            o_ref[:] = res[0:_G, :]


_fin3_call = pl.pallas_call(
    _fin3_body,
    grid=(2, _NBLK),
    in_specs=[
        pl.BlockSpec((_NC, _BLK, _D), lambda p, i: (0, i, 0)),
        pl.BlockSpec((_BLK, _D), lambda p, i: (i, 0)),
        pl.BlockSpec((_BLK, _D), lambda p, i: (i, 0)),
        pl.BlockSpec((1, _D), lambda p, i: (0, 0)),
        pl.BlockSpec((_D, _D), lambda p, i: (0, 0)),
        pl.BlockSpec((1, _D), lambda p, i: (0, 0)),
        pl.BlockSpec((1, _D), lambda p, i: (0, 0)),
        pl.BlockSpec((1, 1), lambda p, i: (0, 0)),
        pl.BlockSpec((_BLK, _D), lambda p, i: (i, 0)),
        pl.BlockSpec((_D, _D), lambda p, i: (0, 0)),
        pl.BlockSpec((1, _D), lambda p, i: (0, 0)),
        pl.BlockSpec((_D, _D), lambda p, i: (0, 0)),
        pl.BlockSpec((1, _D), lambda p, i: (0, 0)),
    ],
    out_specs=pl.BlockSpec((_G, _D), lambda p, i: (0, 0)),
    out_shape=jax.ShapeDtypeStruct((_G, _D), F32),
    scratch_shapes=[
        pltpu.VMEM((1, _D), F32),
        pltpu.VMEM((_D, _D), F32),
        pltpu.VMEM((_D, _D), F32),
    ],
)


def kernel(x, edge_index, batch, W1, b1, W2, b2, W3, b3, g1, be1, g2, be2,
           gW1, gb1, gW2, gb2, mW1, mb1, mW2, mb2):
    e3 = edge_index.reshape(_NC, _NCHROWS, _CH)
    dst = e3[1]
    batchb = jnp.broadcast_to(batch.astype(jnp.int32)[:, None], (_N, _D))
    b1r = b1.reshape(1, _D)
    b2r = b2.reshape(1, _D)
    b3r = b3.reshape(1, _D)
    g1r = g1.reshape(1, _D)
    be1r = be1.reshape(1, _D)
    g2r = g2.reshape(1, _D)
    be2r = be2.reshape(1, _D)
    gb1r = gb1.reshape(1, _D)
    gw2r = gW2.reshape(1, _D)
    gb2r = gb2.reshape(1, 1)
    mb1r = mb1.reshape(1, _D)
    mb2r = mb2.reshape(1, _D)

    degp = _deg_call(dst)
    hs1, dinvb = _pre_call(degp, x, W1)
    s1 = _agg_call(hs1, e3)
    hs2 = _postnext_call(s1, hs1, dinvb, b1r, g1r, be1r, W2)
    s2 = _agg_call(hs2, e3)
    hs3 = _postnext_call(s2, hs2, dinvb, b2r, g2r, be2r, W3)
    s3 = _agg_call(hs3, e3)
    out = _fin3_call(s3, hs3, dinvb, b3r, gW1, gb1r, gw2r, gb2r, batchb,
                     mW1, mb1r, mW2, mb2r)
    return out
